# baseline (speedup 1.0000x reference)
import functools

import jax
import jax.numpy as jnp
from jax import lax
from jax.experimental import pallas as pl
from jax.experimental.pallas import tpu as pltpu
from jax.experimental.pallas import tpu_sc as plsc

BEAM = 32
VOCAB = 1_000_000
L = 2048
CMB = 10
CB = CMB * L
GRID = -(-VOCAB // CB)
NCHP = GRID * 16
CSTRIDE = 1024
CBLOCKS = L // 16
NEG = -3.0e38
BIGI = 2**30


def _rev(x):
    return lax.rev(x, dimensions=(0,))


def _cge(a, ai, b, bi):
    return (a > b) | ((a == b) & (ai <= bi))


def _merge32(state, bv, bi):
    T1, T2, I1, I2 = state
    bs, bis = plsc.sort_key_val(bv, bi, descending=True)
    br, bir = _rev(bs), _rev(bis)
    keep = _cge(T2, I2, br, bir)
    hi = jnp.where(keep, T2, br)
    hii = jnp.where(keep, I2, bir)
    his, hiis = plsc.sort_key_val(hi, hii, descending=True)
    hir, hiir = _rev(his), _rev(hiis)
    k2 = _cge(T1, I1, hir, hiir)
    av = jnp.where(k2, T1, hir)
    ai = jnp.where(k2, I1, hiir)
    bv2 = jnp.where(k2, hir, T1)
    bi2 = jnp.where(k2, hiir, I1)
    T1, I1 = plsc.sort_key_val(av, ai, descending=True)
    T2, I2 = plsc.sort_key_val(bv2, bi2, descending=True)
    return (T1, T2, I1, I2)


def _init32():
    return (jnp.full((16,), NEG, jnp.float32),
            jnp.full((16,), NEG, jnp.float32),
            jnp.full((16,), BIGI, jnp.int32),
            jnp.full((16,), BIGI, jnp.int32))


def _tc_cmax_body(p_ref, o_ref):
    i = pl.program_id(0)
    v = p_ref[...]
    col = i * CB + lax.broadcasted_iota(jnp.int32, (BEAM, CB), 1)
    v = jnp.where(col >= VOCAB - 1, NEG, v)
    parts = [jnp.max(v[:, j * L:(j + 1) * L], axis=-1, keepdims=True)
             for j in range(CMB)]
    parts.append(jnp.full((BEAM, 16 - CMB), NEG, jnp.float32))
    o_ref[...] = jnp.concatenate(parts, axis=1).reshape(1, BEAM, 16)


_tc_cmax = pl.pallas_call(
    _tc_cmax_body,
    grid=(GRID,),
    in_specs=[pl.BlockSpec((BEAM, CB), lambda i: (0, i))],
    out_specs=pl.BlockSpec((1, BEAM, 16), lambda i: (i, 0, 0)),
    out_shape=jax.ShapeDtypeStruct((GRID, BEAM, 16), jnp.float32),
)


def _sc_body(h_hbm, cm_hbm, p_hbm, cv_out, cg_out,
             hv, cmrow, chunkbuf, surv_v, surv_i, stage_f, stage_i):
    core = lax.axis_index("c")
    sub = lax.axis_index("s")
    row = core * 16 + sub
    iota = lax.iota(jnp.int32, 16)

    pltpu.sync_copy(h_hbm, hv)
    h16 = hv[pl.ds(core * 16, 16)]
    hrow = jnp.max(jnp.where(iota == sub, h16, NEG))

    pltpu.sync_copy(cm_hbm.at[pl.ds(row * NCHP, NCHP)], cmrow)

    def p2_body(b, st):
        v = cmrow[pl.ds(b * 16, 16)] + hrow
        ci = row * CSTRIDE + b * 16 + iota
        return _merge32(st, v, ci)
    st = lax.fori_loop(0, NCHP // 16, p2_body, _init32())

    t32 = st[1][15]
    WI1, WI2 = st[2], st[3]

    def handle(widx, st3):
        cid = jnp.maximum(
            jnp.max(jnp.where(iota == widx, WI1, -1)),
            jnp.max(jnp.where(iota == widx - 16, WI2, -1)))
        c = cid % CSTRIDE
        lo = ((c // 16) * CMB + c % 16) * L
        base = jnp.minimum(lo, VOCAB - L)
        pltpu.sync_copy(p_hbm.at[pl.ds(row * VOCAB + base, L)], chunkbuf)

        def cp_body(b, cnt):
            v = chunkbuf[pl.ds(b * 16, 16)]
            col = base + b * 16 + iota
            valid = (col >= lo) & (col <= VOCAB - 2)
            vb = jnp.where(valid, v + hrow, NEG)
            kp = vb >= t32
            keyg = jnp.where(kp, row * VOCAB + col, BIGI)
            sg, sv = plsc.sort_key_val(keyg, vb, descending=False)
            surv_i[pl.ds(cnt, 16)] = sg
            surv_v[pl.ds(cnt, 16)] = sv
            pc = jnp.sum(jnp.where(kp, 1, 0).astype(jnp.int32))
            return jnp.minimum(cnt + pc, L)
        cnt = lax.fori_loop(0, CBLOCKS, cp_body, jnp.int32(0))

        nfull = cnt // 16
        rem = cnt % 16

        def m_body(b, st3):
            return _merge32(st3, surv_v[pl.ds(b * 16, 16)],
                            surv_i[pl.ds(b * 16, 16)])
        st3 = lax.fori_loop(0, nfull, m_body, st3)
        tv = surv_v[pl.ds(nfull * 16, 16)]
        ti = surv_i[pl.ds(nfull * 16, 16)]
        tm = iota < rem
        return _merge32(st3, jnp.where(tm, tv, NEG),
                        jnp.where(tm, ti, BIGI))

    st3 = lax.fori_loop(0, 32, handle, _init32())

    stage_f[pl.ds(0, 16)] = st3[0]
    stage_f[pl.ds(16, 16)] = st3[1]
    stage_i[pl.ds(0, 16)] = st3[2]
    stage_i[pl.ds(16, 16)] = st3[3]
    pltpu.sync_copy(stage_f, cv_out.at[pl.ds(row * 32, 32)])
    pltpu.sync_copy(stage_i, cg_out.at[pl.ds(row * 32, 32)])


_sc_topk = functools.partial(
    pl.kernel,
    out_type=[jax.ShapeDtypeStruct((1024,), jnp.float32),
              jax.ShapeDtypeStruct((1024,), jnp.int32)],
    mesh=plsc.VectorSubcoreMesh(core_axis_name="c", subcore_axis_name="s"),
    compiler_params=pltpu.CompilerParams(needs_layout_passes=False),
    scratch_types=[
        pltpu.VMEM((BEAM,), jnp.float32),
        pltpu.VMEM((NCHP,), jnp.float32),
        pltpu.VMEM((L,), jnp.float32),
        pltpu.VMEM((L + 16,), jnp.float32),
        pltpu.VMEM((L + 16,), jnp.int32),
        pltpu.VMEM((32,), jnp.float32),
        pltpu.VMEM((32,), jnp.int32),
    ],
)(_sc_body)


def _tc_merge_body(cv_ref, cg_ref, s_ref, h_ref, t_ref):
    v = cv_ref[...]
    g = cg_ref[...]
    colj = lax.broadcasted_iota(jnp.int32, (1, 32), 1)
    sv = jnp.zeros((1, 32), jnp.float32)
    sg = jnp.zeros((1, 32), jnp.int32)
    for j in range(32):
        m = jnp.max(v)
        sel = v == m
        gm = jnp.min(jnp.where(sel, g, BIGI))
        sv = jnp.where(colj == j, m, sv)
        sg = jnp.where(colj == j, gm, sg)
        v = jnp.where(g == gm, NEG, v)
    s_ref[...] = sv
    h_ref[...] = sg // VOCAB
    t_ref[...] = sg % VOCAB


_tc_merge = pl.pallas_call(
    _tc_merge_body,
    out_shape=[jax.ShapeDtypeStruct((1, 32), jnp.float32),
               jax.ShapeDtypeStruct((1, 32), jnp.int32),
               jax.ShapeDtypeStruct((1, 32), jnp.int32)],
)


def kernel(hypo_scores, next_token_probs, beam_width):
    del beam_width
    cm = _tc_cmax(next_token_probs)
    cm = cm.transpose(1, 0, 2).reshape(BEAM * NCHP)
    cand_v, cand_g = _sc_topk(hypo_scores, cm,
                              next_token_probs.reshape(-1))
    s, hy, tok = _tc_merge(cand_v.reshape(8, 128), cand_g.reshape(8, 128))
    return s.reshape(BEAM), hy.reshape(BEAM), tok.reshape(BEAM)
